# Initial kernel scaffold; baseline (speedup 1.0000x reference)
import jax
import jax.numpy as jnp
from jax.experimental import pallas as pl


def kernel(user_ids, topic_ids, user_table, topic_table):
    raise NotImplementedError("write your pallas kernel here")


# speedup vs baseline: 12.4033x; 12.4033x over previous
import functools

import jax
import jax.numpy as jnp
from jax import lax
from jax.experimental import pallas as pl
from jax.experimental.pallas import tpu as pltpu
from jax.experimental.pallas import tpu_sc as plsc

B = 16384
L = 50
NUM_USERS = 100000
USER_DIM = 15
MAX_TOKENS = 10000
TOPIC_DIM = 32

NC = 2
NS = 16
NW = NC * NS
RPW = B // NW
CH = 128
NCH = RPW // CH
GR = 16
NGC = CH // GR
IDX_C = 100
NJ = GR * L // IDX_C
UCHUNK = 128
UNJ = RPW // UCHUNK


def _sc_body(tid2d, ids_t, ttable, uids3d, utab16,
             uout, tout,
             idxv, gbuf, cntv, zbuf, row0v, uidx, ubuf, sbuf, sem):
    wid = lax.axis_index("s") * NC + lax.axis_index("c")
    wbase = wid * RPW

    pltpu.sync_copy(ttable.at[0], row0v)
    r0a = row0v[0:16]
    r0b = row0v[16:32]

    pltpu.sync_copy(uids3d.at[wid], uidx)
    ucps = [
        pltpu.async_copy(utab16.at[uidx.at[j]],
                         ubuf.at[pl.ds(j * UCHUNK, UCHUNK), :], sem)
        for j in range(UNJ)
    ]
    for c in ucps:
        c.wait()
    pltpu.sync_copy(ubuf, uout.at[pl.ds(pl.multiple_of(wbase, CH), RPW), :])

    def chunk_body(c, carry):
        cb = wbase + c * CH

        pltpu.sync_copy(ids_t.at[:, pl.ds(pl.multiple_of(cb, CH), CH)], cntv)

        def cnt_body(t, zs):
            return tuple(
                zs[k] + jnp.where(cntv[t, k * 16:(k + 1) * 16] == 0, 1.0, 0.0)
                for k in range(NGC))

        zs = lax.fori_loop(0, L, cnt_body,
                           tuple(jnp.zeros((16,), jnp.float32)
                                 for _ in range(NGC)))
        for k in range(NGC):
            zbuf[k, :] = zs[k]

        def group_body(g, gcarry):
            base = cb + g * GR
            ioff = pl.multiple_of(base * L // IDX_C, NJ)
            pltpu.sync_copy(tid2d.at[pl.ds(ioff, NJ), :], idxv)
            cps = [
                pltpu.async_copy(ttable.at[idxv.at[j]],
                                 gbuf.at[pl.ds(j * IDX_C, IDX_C), :], sem)
                for j in range(NJ)
            ]
            zv = zbuf[g, :]
            denv = jnp.maximum(jnp.float32(L) - zv, 1.0)
            for cp in cps:
                cp.wait()

            for r in range(GR):
                def sum_body(t, acc):
                    a0, a1 = acc
                    return (a0 + gbuf[r * L + t, 0:16],
                            a1 + gbuf[r * L + t, 16:32])

                a0, a1 = lax.fori_loop(
                    0, L, sum_body,
                    (jnp.zeros((16,), jnp.float32),
                     jnp.zeros((16,), jnp.float32)))
                nz = zv[r]
                den = denv[r]
                sbuf[r, 0:16] = (a0 - nz * r0a) / den
                sbuf[r, 16:32] = (a1 - nz * r0b) / den

            pltpu.sync_copy(sbuf, tout.at[pl.ds(pl.multiple_of(base, GR),
                                                GR), :])
            return gcarry

        lax.fori_loop(0, NGC, group_body, 0)
        return carry

    lax.fori_loop(0, NCH, chunk_body, 0)


@functools.partial(
    pl.kernel,
    out_type=(
        jax.ShapeDtypeStruct((B, 16), jnp.float32),
        jax.ShapeDtypeStruct((B, TOPIC_DIM), jnp.float32),
    ),
    mesh=plsc.VectorSubcoreMesh(core_axis_name="c", subcore_axis_name="s"),
    compiler_params=pltpu.CompilerParams(use_tc_tiling_on_sc=False),
    scratch_types=[
        pltpu.VMEM((NJ, IDX_C), jnp.int32),
        pltpu.VMEM((GR * L, TOPIC_DIM), jnp.float32),
        pltpu.VMEM((L, CH), jnp.int32),
        pltpu.VMEM((NGC, 16), jnp.float32),
        pltpu.VMEM((TOPIC_DIM,), jnp.float32),
        pltpu.VMEM((UNJ, UCHUNK), jnp.int32),
        pltpu.VMEM((RPW, 16), jnp.float32),
        pltpu.VMEM((GR, TOPIC_DIM), jnp.float32),
        pltpu.SemaphoreType.DMA,
    ],
)
def _user_model_sc(tid2d, ids_t, ttable, uids3d, utab16, uout, tout,
                   idxv, gbuf, cntv, zbuf, row0v, uidx, ubuf, sbuf, sem):
    _sc_body(tid2d, ids_t, ttable, uids3d, utab16, uout, tout,
             idxv, gbuf, cntv, zbuf, row0v, uidx, ubuf, sbuf, sem)


def kernel(user_ids, topic_ids, user_table, topic_table):
    tid2d = topic_ids.reshape(B * L // IDX_C, IDX_C)
    ids_t = topic_ids.T
    uids3d = user_ids.reshape(NW, UNJ, UCHUNK)
    utab16 = jnp.pad(user_table, ((0, 0), (0, 1)))
    uout, tout = _user_model_sc(tid2d, ids_t, topic_table, uids3d, utab16)
    return jnp.concatenate([uout[:, :USER_DIM], tout], axis=1)
